# baseline (speedup 1.0000x reference)
import jax
import jax.numpy as jnp
from jax import lax
from jax.experimental import pallas as pl
from jax.experimental.pallas import tpu as pltpu
from jax.experimental.pallas import tpu_sc as plsc

_MOM = 0.99
_NC = 2
_NS = 16
_CH = 64
_BC = 1024


def _sc_segsum(z, y2, c):
    n, d = z.shape
    hd = d // _NC
    rps = n // _NS
    nch = rps // _CH
    cps = c // _NS

    nbuf = 4
    nout = nch // nbuf

    def body(z_hbm, y_hbm, sums_out, acc, ybuf, zzero, *rest):
        zbufs = rest[:nbuf]
        gsems = rest[nbuf:2 * nbuf]
        isem = rest[2 * nbuf]
        ysem = rest[2 * nbuf + 1]
        h = lax.axis_index("c")
        s = lax.axis_index("s")
        col0 = pl.multiple_of(h * hd, hd)
        crow0 = pl.multiple_of(s * cps, cps)
        row0 = pl.multiple_of(s * rps, rps)

        def gather(chunk, b):
            off = pl.multiple_of(row0 + chunk * _CH, _CH)
            pltpu.make_async_copy(
                z_hbm.at[pl.ds(off, _CH), pl.ds(col0, hd)],
                zbufs[b], gsems[b]).start()

        def gather_wait(b):
            pltpu.make_async_copy(
                z_hbm.at[pl.ds(row0, _CH), pl.ds(col0, hd)],
                zbufs[b], gsems[b]).wait()

        for b in range(nbuf):
            gather(b, b)

        zrow = jnp.zeros((16,), jnp.float32)

        def zfill(i, carry):
            for u in range(hd // 16):
                zzero[i, pl.ds(u * 16, 16)] = zrow
            return carry

        lax.fori_loop(0, zzero.shape[0], zfill, 0)
        zr = zzero.shape[0]
        pltpu.make_async_copy(
            y_hbm.at[pl.ds(pl.multiple_of(s * nch, nch), nch)], ybuf,
            ysem).start()
        for q in range(cps // zr):
            pltpu.make_async_copy(
                zzero, acc.at[pl.ds(crow0 + q * zr, zr)], isem).start()
        for q in range(cps // zr):
            pltpu.make_async_copy(
                zzero, acc.at[pl.ds(crow0, zr)], isem).wait()
        pltpu.make_async_copy(
            y_hbm.at[pl.ds(pl.multiple_of(s * nch, nch), nch)], ybuf,
            ysem).wait()
        plsc.subcore_barrier()

        def outer_step(t, carry):
            for b in range(nbuf):
                c_ = t * nbuf + b
                gather_wait(b)
                pltpu.sync_copy(zbufs[b], acc.at[ybuf.at[c_]], add=True)

                @pl.when(t < nout - 1)
                def _():
                    gather(c_ + nbuf, b)

            return carry

        lax.fori_loop(0, nout, outer_step, 0)
        plsc.subcore_barrier()
        pltpu.sync_copy(acc.at[pl.ds(crow0, cps)],
                        sums_out.at[pl.ds(crow0, cps), pl.ds(col0, hd)])

    mesh = plsc.VectorSubcoreMesh(core_axis_name="c", subcore_axis_name="s")
    fn = pl.kernel(
        body,
        mesh=mesh,
        out_type=jax.ShapeDtypeStruct((c, d), jnp.float32),
        scratch_types=[
            pltpu.VMEM_SHARED((c, hd), jnp.float32),
            pltpu.VMEM((nch, _CH), jnp.int32),
            pltpu.VMEM((_CH, hd), jnp.float32),
        ] + [pltpu.VMEM((_CH, hd), jnp.float32)] * nbuf
          + [pltpu.SemaphoreType.DMA] * (nbuf + 2),
    )
    return fn(z, y2)


def _finish_body(sums_ref, protos_ref, mask_ref, out_ref):
    sums = sums_ref[...]
    protos = protos_ref[...]
    mask = mask_ref[...]
    nrm = jnp.sqrt(jnp.sum(sums * sums, axis=1, keepdims=True))
    zc = sums / jnp.maximum(nrm, 1e-12)
    ema = _MOM * protos + (1.0 - _MOM) * zc
    enrm = jnp.sqrt(jnp.sum(ema * ema, axis=1, keepdims=True))
    ema = ema / jnp.maximum(enrm, 1e-12)
    new = jnp.where(mask > 0.0, ema, zc)
    out_ref[...] = jnp.where(nrm > 0.0, new, protos)


def kernel(z, y, protos, init_mask):
    n, d = z.shape
    c = protos.shape[0]
    y2 = y.reshape(n // _CH, _CH).astype(jnp.int32)
    sums = _sc_segsum(z, y2, c)

    maskf = init_mask.astype(jnp.float32).reshape(c, 1)
    out = pl.pallas_call(
        _finish_body,
        grid=(c // _BC,),
        in_specs=[
            pl.BlockSpec((_BC, d), lambda i: (i, 0)),
            pl.BlockSpec((_BC, d), lambda i: (i, 0)),
            pl.BlockSpec((_BC, 1), lambda i: (i, 0)),
        ],
        out_specs=pl.BlockSpec((_BC, d), lambda i: (i, 0)),
        out_shape=jax.ShapeDtypeStruct((c, d), jnp.float32),
    )(sums, protos, maskf)
    return out


# speedup vs baseline: 1.0227x; 1.0227x over previous
import jax
import jax.numpy as jnp
from jax import lax
from jax.experimental import pallas as pl
from jax.experimental.pallas import tpu as pltpu
from jax.experimental.pallas import tpu_sc as plsc

_MOM = 0.99
_NC = 2
_NS = 16
_CH = 64
_BC = 4096


def _sc_segsum(z, y2, c):
    n, d = z.shape
    hd = d // _NC
    rps = n // _NS
    nch = rps // _CH
    cps = c // _NS

    nbuf = 4
    nout = nch // nbuf

    def body(z_hbm, y_hbm, sums_out, acc, ybuf, zzero, *rest):
        zbufs = rest[:nbuf]
        gsems = rest[nbuf:2 * nbuf]
        isem = rest[2 * nbuf]
        ysem = rest[2 * nbuf + 1]
        h = lax.axis_index("c")
        s = lax.axis_index("s")
        col0 = pl.multiple_of(h * hd, hd)
        crow0 = pl.multiple_of(s * cps, cps)
        row0 = pl.multiple_of(s * rps, rps)

        def gather(chunk, b):
            off = pl.multiple_of(row0 + chunk * _CH, _CH)
            pltpu.make_async_copy(
                z_hbm.at[pl.ds(off, _CH), pl.ds(col0, hd)],
                zbufs[b], gsems[b]).start()

        def gather_wait(b):
            pltpu.make_async_copy(
                z_hbm.at[pl.ds(row0, _CH), pl.ds(col0, hd)],
                zbufs[b], gsems[b]).wait()

        for b in range(nbuf):
            gather(b, b)

        zrow = jnp.zeros((16,), jnp.float32)

        def zfill(i, carry):
            for u in range(hd // 16):
                zzero[i, pl.ds(u * 16, 16)] = zrow
            return carry

        lax.fori_loop(0, zzero.shape[0], zfill, 0)
        zr = zzero.shape[0]
        pltpu.make_async_copy(
            y_hbm.at[pl.ds(pl.multiple_of(s * nch, nch), nch)], ybuf,
            ysem).start()
        for q in range(cps // zr):
            pltpu.make_async_copy(
                zzero, acc.at[pl.ds(crow0 + q * zr, zr)], isem).start()
        for q in range(cps // zr):
            pltpu.make_async_copy(
                zzero, acc.at[pl.ds(crow0, zr)], isem).wait()
        pltpu.make_async_copy(
            y_hbm.at[pl.ds(pl.multiple_of(s * nch, nch), nch)], ybuf,
            ysem).wait()
        plsc.subcore_barrier()

        def outer_step(t, carry):
            for b in range(nbuf):
                c_ = t * nbuf + b
                gather_wait(b)
                pltpu.sync_copy(zbufs[b], acc.at[ybuf.at[c_]], add=True)

                @pl.when(t < nout - 1)
                def _():
                    gather(c_ + nbuf, b)

            return carry

        lax.fori_loop(0, nout, outer_step, 0)
        plsc.subcore_barrier()
        pltpu.sync_copy(acc.at[pl.ds(crow0, cps)],
                        sums_out.at[pl.ds(crow0, cps), pl.ds(col0, hd)])

    mesh = plsc.VectorSubcoreMesh(core_axis_name="c", subcore_axis_name="s")
    fn = pl.kernel(
        body,
        mesh=mesh,
        out_type=jax.ShapeDtypeStruct((c, d), jnp.float32),
        scratch_types=[
            pltpu.VMEM_SHARED((c, hd), jnp.float32),
            pltpu.VMEM((nch, _CH), jnp.int32),
            pltpu.VMEM((_CH, hd), jnp.float32),
        ] + [pltpu.VMEM((_CH, hd), jnp.float32)] * nbuf
          + [pltpu.SemaphoreType.DMA] * (nbuf + 2),
    )
    return fn(z, y2)


def _finish_body(sums_ref, protos_ref, mask_ref, out_ref):
    sums = sums_ref[...]
    protos = protos_ref[...]
    mask = mask_ref[...]
    nrm = jnp.sqrt(jnp.sum(sums * sums, axis=1, keepdims=True))
    zc = sums / jnp.maximum(nrm, 1e-12)
    ema = _MOM * protos + (1.0 - _MOM) * zc
    enrm = jnp.sqrt(jnp.sum(ema * ema, axis=1, keepdims=True))
    ema = ema / jnp.maximum(enrm, 1e-12)
    new = jnp.where(mask > 0.0, ema, zc)
    out_ref[...] = jnp.where(nrm > 0.0, new, protos)


def kernel(z, y, protos, init_mask):
    n, d = z.shape
    c = protos.shape[0]
    y2 = y.reshape(n // _CH, _CH).astype(jnp.int32)
    sums = _sc_segsum(z, y2, c)

    maskf = init_mask.astype(jnp.float32).reshape(c, 1)
    out = pl.pallas_call(
        _finish_body,
        grid=(c // _BC,),
        in_specs=[
            pl.BlockSpec((_BC, d), lambda i: (i, 0)),
            pl.BlockSpec((_BC, d), lambda i: (i, 0)),
            pl.BlockSpec((_BC, 1), lambda i: (i, 0)),
        ],
        out_specs=pl.BlockSpec((_BC, d), lambda i: (i, 0)),
        out_shape=jax.ShapeDtypeStruct((c, d), jnp.float32),
    )(sums, protos, maskf)
    return out
